# baseline (speedup 1.0000x reference)
import jax
import jax.numpy as jnp
from jax.experimental import pallas as pl

NUM_OBS = 100000
DIM = 128
BLOCK_ROWS = 1000


def _expmap0_block(x_ref, o_ref):
    u = x_ref[...]
    sq = jnp.sum(u * u, axis=1, keepdims=True)
    nrm = jnp.maximum(jnp.sqrt(sq), 1e-15)
    o_ref[...] = u * (jnp.tanh(nrm) / nrm)


def kernel(x):
    grid = (NUM_OBS // BLOCK_ROWS,)
    return pl.pallas_call(
        _expmap0_block,
        grid=grid,
        in_specs=[pl.BlockSpec((BLOCK_ROWS, DIM), lambda i: (i, 0))],
        out_specs=pl.BlockSpec((BLOCK_ROWS, DIM), lambda i: (i, 0)),
        out_shape=jax.ShapeDtypeStruct((NUM_OBS, DIM), x.dtype),
    )(x)


# speedup vs baseline: 1.8567x; 1.8567x over previous
import jax
import jax.numpy as jnp
from jax.experimental import pallas as pl

NUM_OBS = 100000
DIM = 128
BLOCK_ROWS = 4000


def _expmap0_block(x_ref, o_ref):
    u = x_ref[...]
    sq = jnp.sum(u * u, axis=1, keepdims=True)
    nrm = jnp.maximum(jnp.sqrt(sq), 1e-15)
    o_ref[...] = u * (jnp.tanh(nrm) / nrm)


def kernel(x):
    grid = (NUM_OBS // BLOCK_ROWS,)
    return pl.pallas_call(
        _expmap0_block,
        grid=grid,
        in_specs=[pl.BlockSpec((BLOCK_ROWS, DIM), lambda i: (i, 0))],
        out_specs=pl.BlockSpec((BLOCK_ROWS, DIM), lambda i: (i, 0)),
        out_shape=jax.ShapeDtypeStruct((NUM_OBS, DIM), x.dtype),
    )(x)


# speedup vs baseline: 2.2172x; 1.1942x over previous
import jax
import jax.numpy as jnp
from jax.experimental import pallas as pl

NUM_OBS = 100000
DIM = 128
BLOCK_ROWS = 10000


def _expmap0_block(x_ref, o_ref):
    u = x_ref[...]
    sq = jnp.sum(u * u, axis=1, keepdims=True)
    nrm = jnp.maximum(jnp.sqrt(sq), 1e-15)
    o_ref[...] = u * (jnp.tanh(nrm) / nrm)


def kernel(x):
    grid = (NUM_OBS // BLOCK_ROWS,)
    return pl.pallas_call(
        _expmap0_block,
        grid=grid,
        in_specs=[pl.BlockSpec((BLOCK_ROWS, DIM), lambda i: (i, 0))],
        out_specs=pl.BlockSpec((BLOCK_ROWS, DIM), lambda i: (i, 0)),
        out_shape=jax.ShapeDtypeStruct((NUM_OBS, DIM), x.dtype),
    )(x)


# speedup vs baseline: 2.2232x; 1.0027x over previous
import jax
import jax.numpy as jnp
from jax.experimental import pallas as pl

NUM_OBS = 100000
DIM = 128
BLOCK_ROWS = 20000


def _expmap0_block(x_ref, o_ref):
    u = x_ref[...]
    sq = jnp.sum(u * u, axis=1, keepdims=True)
    nrm = jnp.maximum(jnp.sqrt(sq), 1e-15)
    o_ref[...] = u * (jnp.tanh(nrm) / nrm)


def kernel(x):
    grid = (NUM_OBS // BLOCK_ROWS,)
    return pl.pallas_call(
        _expmap0_block,
        grid=grid,
        in_specs=[pl.BlockSpec((BLOCK_ROWS, DIM), lambda i: (i, 0))],
        out_specs=pl.BlockSpec((BLOCK_ROWS, DIM), lambda i: (i, 0)),
        out_shape=jax.ShapeDtypeStruct((NUM_OBS, DIM), x.dtype),
    )(x)
